# baseline (speedup 1.0000x reference)
import functools
import math

import jax
import jax.numpy as jnp
from jax import lax
from jax.experimental import pallas as pl
from jax.experimental.pallas import tpu as pltpu
from jax.experimental.pallas import tpu_sc as plsc

H = 8


def _qkv_body(scale, xb, wq, wk, wv, bq, bk, bv, qo, kvo):
    x = xb[...]
    dn = (((1,), (1,)), ((), ()))
    qo[...] = lax.dot_general(x, wq[...], dn, preferred_element_type=jnp.float32) + bq[0]
    k = (lax.dot_general(x, wk[...], dn, preferred_element_type=jnp.float32) + bk[0]) * scale
    v = lax.dot_general(x, wv[...], dn, preferred_element_type=jnp.float32) + bv[0]
    kvo[...] = jnp.concatenate([k, v], axis=1)


def _qkv_proj(x, Wq, bq, Wk, bk, Wv, bv, scale):
    N, D = x.shape
    Dh = D // 2
    B = 1000
    nb = N // B
    w_spec = pl.BlockSpec((Dh, D), lambda i, h: (h, 0))
    b_spec = pl.BlockSpec((1, 1, Dh), lambda i, h: (h, 0, 0))
    return pl.pallas_call(
        functools.partial(_qkv_body, scale),
        grid=(nb, 2),
        in_specs=[
            pl.BlockSpec((B, D), lambda i, h: (i, 0)),
            w_spec, w_spec, w_spec, b_spec, b_spec, b_spec,
        ],
        out_specs=[
            pl.BlockSpec((B, Dh), lambda i, h: (h * nb + i, 0)),
            pl.BlockSpec((B, D), lambda i, h: (h * nb + i, 0)),
        ],
        out_shape=[
            jax.ShapeDtypeStruct((2 * N, Dh), jnp.float32),
            jax.ShapeDtypeStruct((2 * N, D), jnp.float32),
        ],
    )(x, Wq, Wk, Wv, bq.reshape(2, 1, Dh), bk.reshape(2, 1, Dh), bv.reshape(2, 1, Dh))


def _pack_body(ei, out):
    e = ei[...]
    out[...] = jnp.bitwise_or(jnp.left_shift(e[1], 16), e[0])


def _pack_edges(edge_index):
    E = edge_index.shape[1]
    R, W = 2000, E // 2000
    BR = 200
    ein = edge_index.reshape(2, R, W)
    return pl.pallas_call(
        _pack_body,
        grid=(R // BR,),
        in_specs=[pl.BlockSpec((2, BR, W), lambda i: (0, i, 0))],
        out_specs=pl.BlockSpec((BR, W), lambda i: (i, 0)),
        out_shape=jax.ShapeDtypeStruct((R, W), jnp.int32),
    )(ein).reshape(E)


def _edge_body(nodes_n, chunk_c, chunks_n,
               q2, kv2, pk2, aout,
               pka, ki0, ki1, qi0, qi1, ds0, ds1,
               kv0, kv1, qb0, qb1, m0, m1,
               acc, gs0, gs1, ss0, ss1):
    N = nodes_n
    C = chunk_c
    D = kv0.shape[1]
    Dh = qb0.shape[1]
    NW = 10
    rows_per = N // NW

    c = lax.axis_index("c")
    s = lax.axis_index("s")
    cN = c * N

    kvb = (kv0, kv1)
    qb = (qb0, qb1)
    mb = (m0, m1)
    ki = (ki0, ki1)
    qi = (qi0, qi1)
    dsc = (ds0, ds1)
    gs = (gs0, gs1)
    ss = (ss0, ss1)

    pltpu.sync_copy(pk2.at[s], pka)

    def zfill(i, _):
        for j in range(D // 16):
            m0[i, pl.ds(j * 16, 16)] = jnp.zeros((16,), jnp.float32)
        return 0
    lax.fori_loop(0, C, zfill, 0)

    @pl.when(s < NW)
    def _zero():
        base = s * rows_per
        for r in range(rows_per // C):
            pltpu.sync_copy(m0, acc.at[pl.ds(base + r * C, C)])
        rem = rows_per % C
        if rem:
            pltpu.sync_copy(m0.at[pl.ds(0, rem)],
                            acc.at[pl.ds(base + rows_per - rem, rem)])
    plsc.subcore_barrier()

    def issue(ch, b):
        for j in range(C // 16):
            sl = pl.ds(j * 16, 16)
            pe = pka[pl.ds(ch * C + j * 16, 16)]
            ki[b][sl] = jnp.bitwise_and(pe, 0xFFFF) + cN
            qi[b][sl] = jnp.right_shift(pe, 16) + cN
        pltpu.async_copy(kv2.at[ki[b]], kvb[b], gs[b])
        pltpu.async_copy(q2.at[qi[b]], qb[b], gs[b])

    def slot_work(g, b, first, last):
        if not first:
            @pl.when(g >= 2)
            def _drain_scatter():
                pltpu.make_async_copy(mb[b], acc.at[dsc[b]], ss[b]).wait()
        pltpu.make_async_copy(kv2.at[ki[b]], kvb[b], gs[b]).wait()
        pltpu.make_async_copy(q2.at[qi[b]], qb[b], gs[b]).wait()

        def edge(e, _):
            for j in range(Dh // 16):
                sl = pl.ds(j * 16, 16)
                sl2 = pl.ds(Dh + j * 16, 16)
                mm = jnp.exp(kvb[b][e, sl] * qb[b][e, sl])
                mb[b][e, sl] = mm
                mb[b][e, sl2] = mm * kvb[b][e, sl2]
            return 0
        lax.fori_loop(0, C, edge, 0)

        for j in range(C // 16):
            sl = pl.ds(j * 16, 16)
            pe = pka[pl.ds(g * C + j * 16, 16)]
            dsc[b][sl] = jnp.right_shift(pe, 16)
        pltpu.async_copy(mb[b], acc.at[dsc[b]], ss[b], add=True)

        if not last:
            @pl.when(g + 2 < chunks_n)
            def _prefetch():
                issue(g + 2, b)

    issue(0, 0)
    issue(1, 1)

    def pipe(i, _):
        slot_work(2 * i, 0, False, False)
        slot_work(2 * i + 1, 1, False, False)
        return 0
    lax.fori_loop(0, (chunks_n - 1) // 2, pipe, 0)
    slot_work(chunks_n - 1, 0, False, True)

    pltpu.make_async_copy(mb[0], acc.at[dsc[0]], ss[0]).wait()
    pltpu.make_async_copy(mb[1], acc.at[dsc[1]], ss[1]).wait()
    plsc.subcore_barrier()

    @pl.when(s < NW)
    def _writeout():
        wbase = s * rows_per
        pltpu.sync_copy(acc.at[pl.ds(wbase, rows_per)],
                        aout.at[pl.ds(cN + wbase, rows_per)])


def _edge_pass(q2, kv2, packed, N, D):
    E = packed.shape[0]
    NS = 16
    Es = E // NS
    C = 32
    nch = Es // C
    Dh = D // 2
    mesh = plsc.VectorSubcoreMesh(core_axis_name="c", subcore_axis_name="s")
    f = pl.kernel(
        functools.partial(_edge_body, N, C, nch),
        out_type=jax.ShapeDtypeStruct((2 * N, D), jnp.float32),
        mesh=mesh,
        scratch_types=[
            pltpu.VMEM((Es,), jnp.int32),
            pltpu.VMEM((C,), jnp.int32),
            pltpu.VMEM((C,), jnp.int32),
            pltpu.VMEM((C,), jnp.int32),
            pltpu.VMEM((C,), jnp.int32),
            pltpu.VMEM((C,), jnp.int32),
            pltpu.VMEM((C,), jnp.int32),
            pltpu.VMEM((C, D), jnp.float32),
            pltpu.VMEM((C, D), jnp.float32),
            pltpu.VMEM((C, Dh), jnp.float32),
            pltpu.VMEM((C, Dh), jnp.float32),
            pltpu.VMEM((C, D), jnp.float32),
            pltpu.VMEM((C, D), jnp.float32),
            pltpu.VMEM_SHARED((N, D), jnp.float32),
            pltpu.SemaphoreType.DMA,
            pltpu.SemaphoreType.DMA,
            pltpu.SemaphoreType.DMA,
            pltpu.SemaphoreType.DMA,
        ],
        compiler_params=pltpu.CompilerParams(use_tc_tiling_on_sc=False),
    )
    return f(q2, kv2, packed.reshape(NS, Es))


def _out_body(alo, ahi, wo, bo, out):
    Dh = alo.shape[2] // 2
    al = alo[0]
    ah = ahi[0]
    zl = al[:, :Dh]
    zh = ah[:, :Dh]
    rl = al[:, Dh:] / jnp.where(zl == 0.0, 1.0, zl)
    rh = ah[:, Dh:] / jnp.where(zh == 0.0, 1.0, zh)
    r = jnp.concatenate([rl, rh], axis=1)
    dn = (((1,), (1,)), ((), ()))
    out[...] = lax.dot_general(r, wo[...], dn, preferred_element_type=jnp.float32) + bo[0]


def _out_proj(A, Wo, bo, N, D):
    B = 1000
    nb = N // B
    a3 = A.reshape(2, N, D)
    return pl.pallas_call(
        _out_body,
        grid=(nb,),
        in_specs=[
            pl.BlockSpec((1, B, D), lambda i: (0, i, 0)),
            pl.BlockSpec((1, B, D), lambda i: (1, i, 0)),
            pl.BlockSpec((D, D), lambda i: (0, 0)),
            pl.BlockSpec((1, D), lambda i: (0, 0)),
        ],
        out_specs=pl.BlockSpec((B, D), lambda i: (i, 0)),
        out_shape=jax.ShapeDtypeStruct((N, D), jnp.float32),
    )(a3, a3, Wo, bo.reshape(1, D))


def kernel(x, edge_index, Wq, bq, Wk, bk, Wv, bv, Wo, bo):
    N, D = x.shape
    dk = D // H
    scale = 1.0 / math.sqrt(dk)
    q2, kv2 = _qkv_proj(x, Wq, bq, Wk, bk, Wv, bv, scale)
    packed = _pack_edges(edge_index)
    A = _edge_pass(q2, kv2, packed, N, D)
    return _out_proj(A, Wo, bo, N, D)


# speedup vs baseline: 3.0716x; 3.0716x over previous
import functools
import math

import jax
import jax.numpy as jnp
from jax import lax
from jax.experimental import pallas as pl
from jax.experimental.pallas import tpu as pltpu
from jax.experimental.pallas import tpu_sc as plsc

H = 8


def _qkv_body(scale, xb, wq, wk, wv, bq, bk, bv, ei, qo, kvo, po):
    x = xb[...]
    dn = (((1,), (1,)), ((), ()))
    q = lax.dot_general(x, wq[...], dn, preferred_element_type=jnp.float32) + bq[0]
    qo[...] = q.astype(jnp.bfloat16)
    k = (lax.dot_general(x, wk[...], dn, preferred_element_type=jnp.float32) + bk[0]) * scale
    v = lax.dot_general(x, wv[...], dn, preferred_element_type=jnp.float32) + bv[0]
    kvo[...] = jnp.concatenate([k, v], axis=1).astype(jnp.bfloat16)
    e = ei[...]
    po[...] = jnp.bitwise_or(jnp.left_shift(e[1], 16), e[0])


def _qkv_proj(x, edge_index, Wq, bq, Wk, bk, Wv, bv, scale):
    N, D = x.shape
    Dh = D // 2
    B = 1000
    nb = N // B
    E = edge_index.shape[1]
    R, W = 2000, E // 2000
    BR = R // nb
    ein = edge_index.reshape(2, R, W)
    w_spec = pl.BlockSpec((Dh, D), lambda i, h: (h, 0))
    b_spec = pl.BlockSpec((1, 1, Dh), lambda i, h: (h, 0, 0))
    q2, kv2, p2 = pl.pallas_call(
        functools.partial(_qkv_body, scale),
        grid=(nb, 2),
        in_specs=[
            pl.BlockSpec((B, D), lambda i, h: (i, 0)),
            w_spec, w_spec, w_spec, b_spec, b_spec, b_spec,
            pl.BlockSpec((2, BR, W), lambda i, h: (0, i, 0)),
        ],
        out_specs=[
            pl.BlockSpec((B, Dh), lambda i, h: (h * nb + i, 0)),
            pl.BlockSpec((B, D), lambda i, h: (h * nb + i, 0)),
            pl.BlockSpec((BR, W), lambda i, h: (i, 0)),
        ],
        out_shape=[
            jax.ShapeDtypeStruct((2 * N, Dh), jnp.bfloat16),
            jax.ShapeDtypeStruct((2 * N, D), jnp.bfloat16),
            jax.ShapeDtypeStruct((R, W), jnp.int32),
        ],
    )(x, Wq, Wk, Wv, bq.reshape(2, 1, Dh), bk.reshape(2, 1, Dh),
      bv.reshape(2, 1, Dh), ein)
    return q2, kv2, p2.reshape(E)


def _edge_body(nodes_n, chunk_c, chunks_n,
               q2, kv2, pk3, aout,
               pk0, pk1, pk2_, ki0, ki1, ki2, qi0, qi1, qi2,
               ds0, ds1, kv0, kv1, kv2_, qb0, qb1, qb2, ob0, ob1,
               acc, is0, is1, is2, gs0, gs1, gs2, ss0, ss1):
    N = nodes_n
    C = chunk_c
    D = ob0.shape[1]
    NW = 10
    rows_per = N // NW

    c = lax.axis_index("c")
    s = lax.axis_index("s")
    cN = c * N

    pkc = (pk0, pk1, pk2_)
    ki = (ki0, ki1, ki2)
    qi = (qi0, qi1, qi2)
    dsc = (ds0, ds1)
    kvb = (kv0, kv1, kv2_)
    qb = (qb0, qb1, qb2)
    ob = (ob0, ob1)
    isem = (is0, is1, is2)
    gs = (gs0, gs1, gs2)
    ss = (ss0, ss1)

    def zfill(i, _):
        for j in range(D // 16):
            ob0[i, pl.ds(j * 16, 16)] = jnp.zeros((16,), jnp.float32)
        return 0
    lax.fori_loop(0, C, zfill, 0)

    @pl.when(s < NW)
    def _zero():
        base = s * rows_per
        for r in range(rows_per // C):
            pltpu.sync_copy(ob0, acc.at[pl.ds(base + r * C, C)])
        rem = rows_per % C
        if rem:
            pltpu.sync_copy(ob0.at[pl.ds(0, rem)],
                            acc.at[pl.ds(base + rows_per - rem, rem)])
    plsc.subcore_barrier()

    def issue_rows(ch, b):
        for j in range(C // 16):
            sl = pl.ds(j * 16, 16)
            pe = pkc[b][sl]
            ki[b][sl] = jnp.bitwise_and(pe, 0xFFFF) + cN
            qi[b][sl] = jnp.right_shift(pe, 16) + cN
        pltpu.async_copy(kv2.at[ki[b]], kvb[b], gs[b])
        pltpu.async_copy(q2.at[qi[b]], qb[b], gs[b])

    himask = jnp.int32(-65536)

    def expand(w):
        ev = plsc.bitcast(jnp.left_shift(w, 16), jnp.float32)
        od = plsc.bitcast(jnp.bitwise_and(w, himask), jnp.float32)
        return ev, od

    def slot_work(g, b, b2):
        pltpu.make_async_copy(kv2.at[ki[b]], kvb[b], gs[b]).wait()
        pltpu.make_async_copy(q2.at[qi[b]], qb[b], gs[b]).wait()

        @plsc.parallel_loop(0, C, unroll=8)
        def edge(e):
            for grp in range(D // 64):
                kw = plsc.bitcast(kvb[b][e, pl.ds(grp * 32, 32)], jnp.int32)
                qw = plsc.bitcast(qb[b][e, pl.ds(grp * 32, 32)], jnp.int32)
                vw = plsc.bitcast(kvb[b][e, pl.ds(D // 2 + grp * 32, 32)], jnp.int32)
                ke, ko = expand(kw)
                qe, qo = expand(qw)
                ve, vo = expand(vw)
                me = jnp.exp(ke * qe)
                mo = jnp.exp(ko * qo)
                ob[b2][e, pl.ds(grp * 32, 16)] = me
                ob[b2][e, pl.ds(grp * 32 + 16, 16)] = mo
                ob[b2][e, pl.ds(D // 2 + grp * 32, 16)] = me * ve
                ob[b2][e, pl.ds(D // 2 + grp * 32 + 16, 16)] = mo * vo

        for j in range(C // 16):
            sl = pl.ds(j * 16, 16)
            dsc[b2][sl] = jnp.right_shift(pkc[b][sl], 16)
        pltpu.async_copy(ob[b2], acc.at[dsc[b2]], ss[b2], add=True)

        pb = (b2 + 1) % 2
        @pl.when(g >= 1)
        def _drain_scatter():
            pltpu.make_async_copy(ob[pb], acc.at[dsc[pb]], ss[pb]).wait()

        nb = (b + 2) % 3
        @pl.when(g + 2 < chunks_n)
        def _rows_ahead():
            @pl.when(g >= 1)
            def _drain_idx():
                pltpu.make_async_copy(pk3.at[s, 0], pkc[nb], isem[nb]).wait()
            issue_rows(g + 2, nb)

        @pl.when(g + 3 < chunks_n)
        def _idx_ahead():
            pltpu.async_copy(pk3.at[s, g + 3], pkc[b], isem[b])

    pltpu.sync_copy(pk3.at[s, 0], pk0)
    pltpu.sync_copy(pk3.at[s, 1], pk1)
    pltpu.sync_copy(pk3.at[s, 2], pk2_)
    issue_rows(0, 0)
    issue_rows(1, 1)

    def pipe(i, _):
        for t in range(6):
            slot_work(6 * i + t, t % 3, t % 2)
        return 0
    lax.fori_loop(0, chunks_n // 6, pipe, 0)
    for g in range(chunks_n - chunks_n % 6, chunks_n):
        slot_work(g, g % 3, g % 2)

    pltpu.make_async_copy(ob[(chunks_n - 1) % 2],
                          acc.at[dsc[(chunks_n - 1) % 2]],
                          ss[(chunks_n - 1) % 2]).wait()
    plsc.subcore_barrier()

    @pl.when(s < NW)
    def _writeout():
        wbase = s * rows_per
        pltpu.sync_copy(acc.at[pl.ds(wbase, rows_per)],
                        aout.at[pl.ds(cN + wbase, rows_per)])


def _edge_pass(q2, kv2, packed, N, D):
    E = packed.shape[0]
    NS = 16
    Es = E // NS
    C = 80
    nch = Es // C
    Dh = D // 2
    mesh = plsc.VectorSubcoreMesh(core_axis_name="c", subcore_axis_name="s")
    idx = pltpu.VMEM((C,), jnp.int32)
    f = pl.kernel(
        functools.partial(_edge_body, N, C, nch),
        out_type=jax.ShapeDtypeStruct((2 * N, D), jnp.float32),
        mesh=mesh,
        scratch_types=[
            idx, idx, idx,
            idx, idx, idx,
            idx, idx, idx,
            idx, idx,
            pltpu.VMEM((C, D), jnp.bfloat16),
            pltpu.VMEM((C, D), jnp.bfloat16),
            pltpu.VMEM((C, D), jnp.bfloat16),
            pltpu.VMEM((C, Dh), jnp.bfloat16),
            pltpu.VMEM((C, Dh), jnp.bfloat16),
            pltpu.VMEM((C, Dh), jnp.bfloat16),
            pltpu.VMEM((C, D), jnp.float32),
            pltpu.VMEM((C, D), jnp.float32),
            pltpu.VMEM_SHARED((N, D), jnp.float32),
            pltpu.SemaphoreType.DMA,
            pltpu.SemaphoreType.DMA,
            pltpu.SemaphoreType.DMA,
            pltpu.SemaphoreType.DMA,
            pltpu.SemaphoreType.DMA,
            pltpu.SemaphoreType.DMA,
            pltpu.SemaphoreType.DMA,
            pltpu.SemaphoreType.DMA,
        ],
        compiler_params=pltpu.CompilerParams(use_tc_tiling_on_sc=False, needs_layout_passes=False),
    )
    return f(q2, kv2, packed.reshape(NS, nch, C))


def _out_body(alo, ahi, wo, bo, out):
    Dh = alo.shape[2] // 2
    al = alo[0]
    ah = ahi[0]
    zl = al[:, :Dh]
    zh = ah[:, :Dh]
    rl = al[:, Dh:] / jnp.where(zl == 0.0, 1.0, zl)
    rh = ah[:, Dh:] / jnp.where(zh == 0.0, 1.0, zh)
    r = jnp.concatenate([rl, rh], axis=1)
    dn = (((1,), (1,)), ((), ()))
    out[...] = lax.dot_general(r, wo[...], dn, preferred_element_type=jnp.float32) + bo[0]


def _out_proj(A, Wo, bo, N, D):
    B = 1000
    nb = N // B
    a3 = A.reshape(2, N, D)
    return pl.pallas_call(
        _out_body,
        grid=(nb,),
        in_specs=[
            pl.BlockSpec((1, B, D), lambda i: (0, i, 0)),
            pl.BlockSpec((1, B, D), lambda i: (1, i, 0)),
            pl.BlockSpec((D, D), lambda i: (0, 0)),
            pl.BlockSpec((1, D), lambda i: (0, 0)),
        ],
        out_specs=pl.BlockSpec((B, D), lambda i: (i, 0)),
        out_shape=jax.ShapeDtypeStruct((N, D), jnp.float32),
    )(a3, a3, Wo, bo.reshape(1, D))


def kernel(x, edge_index, Wq, bq, Wk, bk, Wv, bv, Wo, bo):
    N, D = x.shape
    dk = D // H
    scale = 1.0 / math.sqrt(dk)
    q2, kv2, packed = _qkv_proj(x, edge_index, Wq, bq, Wk, bk, Wv, bv, scale)
    A = _edge_pass(q2, kv2, packed, N, D)
    perm64 = [g * 32 + u for g in range(2) for u in
              list(range(0, 32, 2)) + list(range(1, 32, 2))]
    perm = jnp.array(perm64 + [64 + p for p in perm64], dtype=jnp.int32)
    return _out_proj(A, Wo[:, perm], bo, N, D)


# speedup vs baseline: 3.5884x; 1.1682x over previous
import functools
import math

import jax
import jax.numpy as jnp
from jax import lax
from jax.experimental import pallas as pl
from jax.experimental.pallas import tpu as pltpu
from jax.experimental.pallas import tpu_sc as plsc

H = 8


def _qkv_body(scale, xb, wq, wk, wv, bq, bk, bv, ei, qo, kvo, po):
    x = xb[...]
    dn = (((1,), (1,)), ((), ()))
    q = lax.dot_general(x, wq[...], dn, preferred_element_type=jnp.float32) + bq[0]
    qo[...] = q.astype(jnp.bfloat16)
    k = (lax.dot_general(x, wk[...], dn, preferred_element_type=jnp.float32) + bk[0]) * scale
    v = lax.dot_general(x, wv[...], dn, preferred_element_type=jnp.float32) + bv[0]
    kvo[...] = jnp.concatenate([k, v], axis=1).astype(jnp.bfloat16)
    e = ei[...]
    po[...] = jnp.bitwise_or(jnp.left_shift(e[1], 16), e[0])


def _qkv_proj(x, edge_index, Wq, bq, Wk, bk, Wv, bv, scale):
    N, D = x.shape
    Dh = D // 2
    B = 1000
    nb = N // B
    E = edge_index.shape[1]
    R, W = 2000, E // 2000
    BR = R // nb
    ein = edge_index.reshape(2, R, W)
    w_spec = pl.BlockSpec((Dh, D), lambda i, h: (h, 0))
    b_spec = pl.BlockSpec((1, 1, Dh), lambda i, h: (h, 0, 0))
    q2, kv2, p2 = pl.pallas_call(
        functools.partial(_qkv_body, scale),
        grid=(nb, 2),
        in_specs=[
            pl.BlockSpec((B, D), lambda i, h: (i, 0)),
            w_spec, w_spec, w_spec, b_spec, b_spec, b_spec,
            pl.BlockSpec((2, BR, W), lambda i, h: (0, i, 0)),
        ],
        out_specs=[
            pl.BlockSpec((B, Dh), lambda i, h: (h * nb + i, 0)),
            pl.BlockSpec((B, D), lambda i, h: (h * nb + i, 0)),
            pl.BlockSpec((BR, W), lambda i, h: (i, 0)),
        ],
        out_shape=[
            jax.ShapeDtypeStruct((2 * N, Dh), jnp.bfloat16),
            jax.ShapeDtypeStruct((2 * N, D), jnp.bfloat16),
            jax.ShapeDtypeStruct((R, W), jnp.int32),
        ],
    )(x, Wq, Wk, Wv, bq.reshape(2, 1, Dh), bk.reshape(2, 1, Dh),
      bv.reshape(2, 1, Dh), ein)
    return q2, kv2, p2.reshape(E)


def _edge_body(nodes_n, chunk_c, chunks_n,
               q2, kv2, pk3, aout,
               pk0, pk1, pk2_, ki0, ki1, ki2, qi0, qi1, qi2,
               ds0, ds1, kv0, kv1, kv2_, qb0, qb1, qb2, ob0, ob1,
               acc, is0, is1, is2, gs0, gs1, gs2, ss0, ss1):
    N = nodes_n
    C = chunk_c
    D = ob0.shape[1]
    NW = 10
    rows_per = N // NW

    c = lax.axis_index("c")
    s = lax.axis_index("s")
    cN = c * N

    pkc = (pk0, pk1, pk2_)
    ki = (ki0, ki1, ki2)
    qi = (qi0, qi1, qi2)
    dsc = (ds0, ds1)
    kvb = (kv0, kv1, kv2_)
    qb = (qb0, qb1, qb2)
    ob = (ob0, ob1)
    isem = (is0, is1, is2)
    gs = (gs0, gs1, gs2)
    ss = (ss0, ss1)

    def zfill(i, _):
        for j in range(D // 16):
            ob0[i, pl.ds(j * 16, 16)] = jnp.zeros((16,), jnp.float32)
        return 0
    lax.fori_loop(0, C, zfill, 0)

    @pl.when(s < NW)
    def _zero():
        base = s * rows_per
        for r in range(rows_per // C):
            pltpu.sync_copy(ob0, acc.at[pl.ds(base + r * C, C)])
        rem = rows_per % C
        if rem:
            pltpu.sync_copy(ob0.at[pl.ds(0, rem)],
                            acc.at[pl.ds(base + rows_per - rem, rem)])
    plsc.subcore_barrier()

    def issue_rows(ch, b):
        for j in range(C // 16):
            sl = pl.ds(j * 16, 16)
            pe = pkc[b][sl]
            ki[b][sl] = jnp.bitwise_and(pe, 0xFFFF) + cN
            qi[b][sl] = jnp.right_shift(pe, 16) + cN
        pltpu.async_copy(kv2.at[ki[b]], kvb[b], gs[b])
        pltpu.async_copy(q2.at[qi[b]], qb[b], gs[b])

    himask = jnp.int32(-65536)

    def expand(w):
        ev = plsc.bitcast(jnp.left_shift(w, 16), jnp.float32)
        od = plsc.bitcast(jnp.bitwise_and(w, himask), jnp.float32)
        return ev, od

    def slot_work(g, b, b2):
        pltpu.make_async_copy(kv2.at[ki[b]], kvb[b], gs[b]).wait()
        pltpu.make_async_copy(q2.at[qi[b]], qb[b], gs[b]).wait()

        @plsc.parallel_loop(0, C, unroll=4)
        def edge(e):
            for grp in range(D // 64):
                kw = plsc.bitcast(kvb[b][e, pl.ds(grp * 32, 32)], jnp.int32)
                qw = plsc.bitcast(qb[b][e, pl.ds(grp * 32, 32)], jnp.int32)
                vw = plsc.bitcast(kvb[b][e, pl.ds(D // 2 + grp * 32, 32)], jnp.int32)
                ke, ko = expand(kw)
                qe, qo = expand(qw)
                ve, vo = expand(vw)
                me = jnp.exp(ke * qe)
                mo = jnp.exp(ko * qo)
                ob[b2][e, pl.ds(grp * 32, 16)] = me
                ob[b2][e, pl.ds(grp * 32 + 16, 16)] = mo
                ob[b2][e, pl.ds(D // 2 + grp * 32, 16)] = me * ve
                ob[b2][e, pl.ds(D // 2 + grp * 32 + 16, 16)] = mo * vo

        for j in range(C // 16):
            sl = pl.ds(j * 16, 16)
            dsc[b2][sl] = jnp.right_shift(pkc[b][sl], 16)
        pltpu.async_copy(ob[b2], acc.at[dsc[b2]], ss[b2], add=True)

        pb = (b2 + 1) % 2
        @pl.when(g >= 1)
        def _drain_scatter():
            pltpu.make_async_copy(ob[pb], acc.at[dsc[pb]], ss[pb]).wait()

        nb = (b + 2) % 3
        @pl.when(g + 2 < chunks_n)
        def _rows_ahead():
            @pl.when(g >= 1)
            def _drain_idx():
                pltpu.make_async_copy(pk3.at[s, 0], pkc[nb], isem[nb]).wait()
            issue_rows(g + 2, nb)

        @pl.when(g + 3 < chunks_n)
        def _idx_ahead():
            pltpu.async_copy(pk3.at[s, g + 3], pkc[b], isem[b])

    pltpu.sync_copy(pk3.at[s, 0], pk0)
    pltpu.sync_copy(pk3.at[s, 1], pk1)
    pltpu.sync_copy(pk3.at[s, 2], pk2_)
    issue_rows(0, 0)
    issue_rows(1, 1)

    def pipe(i, _):
        for t in range(6):
            slot_work(6 * i + t, t % 3, t % 2)
        return 0
    lax.fori_loop(0, chunks_n // 6, pipe, 0)
    for g in range(chunks_n - chunks_n % 6, chunks_n):
        slot_work(g, g % 3, g % 2)

    pltpu.make_async_copy(ob[(chunks_n - 1) % 2],
                          acc.at[dsc[(chunks_n - 1) % 2]],
                          ss[(chunks_n - 1) % 2]).wait()
    plsc.subcore_barrier()

    @pl.when(s < NW)
    def _writeout():
        wbase = s * rows_per
        pltpu.sync_copy(acc.at[pl.ds(wbase, rows_per)],
                        aout.at[pl.ds(cN + wbase, rows_per)])


def _edge_pass(q2, kv2, packed, N, D):
    E = packed.shape[0]
    NS = 16
    Es = E // NS
    C = 80
    nch = Es // C
    Dh = D // 2
    mesh = plsc.VectorSubcoreMesh(core_axis_name="c", subcore_axis_name="s")
    idx = pltpu.VMEM((C,), jnp.int32)
    f = pl.kernel(
        functools.partial(_edge_body, N, C, nch),
        out_type=jax.ShapeDtypeStruct((2 * N, D), jnp.float32),
        mesh=mesh,
        scratch_types=[
            idx, idx, idx,
            idx, idx, idx,
            idx, idx, idx,
            idx, idx,
            pltpu.VMEM((C, D), jnp.bfloat16),
            pltpu.VMEM((C, D), jnp.bfloat16),
            pltpu.VMEM((C, D), jnp.bfloat16),
            pltpu.VMEM((C, Dh), jnp.bfloat16),
            pltpu.VMEM((C, Dh), jnp.bfloat16),
            pltpu.VMEM((C, Dh), jnp.bfloat16),
            pltpu.VMEM((C, D), jnp.float32),
            pltpu.VMEM((C, D), jnp.float32),
            pltpu.VMEM_SHARED((N, D), jnp.float32),
            pltpu.SemaphoreType.DMA,
            pltpu.SemaphoreType.DMA,
            pltpu.SemaphoreType.DMA,
            pltpu.SemaphoreType.DMA,
            pltpu.SemaphoreType.DMA,
            pltpu.SemaphoreType.DMA,
            pltpu.SemaphoreType.DMA,
            pltpu.SemaphoreType.DMA,
        ],
        compiler_params=pltpu.CompilerParams(use_tc_tiling_on_sc=False, needs_layout_passes=False),
    )
    return f(q2, kv2, packed.reshape(NS, nch, C))


def _out_body(alo, ahi, wo, bo, out):
    Dh = alo.shape[2] // 2
    al = alo[0]
    ah = ahi[0]
    zl = al[:, :Dh]
    zh = ah[:, :Dh]
    rl = al[:, Dh:] / jnp.where(zl == 0.0, 1.0, zl)
    rh = ah[:, Dh:] / jnp.where(zh == 0.0, 1.0, zh)
    r = jnp.concatenate([rl, rh], axis=1)
    dn = (((1,), (1,)), ((), ()))
    out[...] = lax.dot_general(r, wo[...], dn, preferred_element_type=jnp.float32) + bo[0]


def _out_proj(A, Wo, bo, N, D):
    B = 1000
    nb = N // B
    a3 = A.reshape(2, N, D)
    return pl.pallas_call(
        _out_body,
        grid=(nb,),
        in_specs=[
            pl.BlockSpec((1, B, D), lambda i: (0, i, 0)),
            pl.BlockSpec((1, B, D), lambda i: (1, i, 0)),
            pl.BlockSpec((D, D), lambda i: (0, 0)),
            pl.BlockSpec((1, D), lambda i: (0, 0)),
        ],
        out_specs=pl.BlockSpec((B, D), lambda i: (i, 0)),
        out_shape=jax.ShapeDtypeStruct((N, D), jnp.float32),
    )(a3, a3, Wo, bo.reshape(1, D))


def kernel(x, edge_index, Wq, bq, Wk, bk, Wv, bv, Wo, bo):
    N, D = x.shape
    dk = D // H
    scale = 1.0 / math.sqrt(dk)
    q2, kv2, packed = _qkv_proj(x, edge_index, Wq, bq, Wk, bk, Wv, bv, scale)
    A = _edge_pass(q2, kv2, packed, N, D)
    perm64 = [g * 32 + u for g in range(2) for u in
              list(range(0, 32, 2)) + list(range(1, 32, 2))]
    perm = jnp.array(perm64 + [64 + p for p in perm64], dtype=jnp.int32)
    return _out_proj(A, Wo[:, perm], bo, N, D)


# speedup vs baseline: 3.6297x; 1.0115x over previous
import functools
import math

import jax
import jax.numpy as jnp
from jax import lax
from jax.experimental import pallas as pl
from jax.experimental.pallas import tpu as pltpu
from jax.experimental.pallas import tpu_sc as plsc

H = 8


def _qkv_body(scale, xb, wq, wk, wv, bq, bk, bv, ei, qo, kvo, po):
    x = xb[...]
    dn = (((1,), (1,)), ((), ()))
    q = lax.dot_general(x, wq[...], dn, preferred_element_type=jnp.float32) + bq[0]
    qo[...] = q.astype(jnp.bfloat16)
    k = (lax.dot_general(x, wk[...], dn, preferred_element_type=jnp.float32) + bk[0]) * scale
    v = lax.dot_general(x, wv[...], dn, preferred_element_type=jnp.float32) + bv[0]
    kvo[...] = jnp.concatenate([k, v], axis=1).astype(jnp.bfloat16)
    e = ei[...]
    po[...] = jnp.bitwise_or(jnp.left_shift(e[1], 16), e[0])


def _qkv_proj(x, edge_index, Wq, bq, Wk, bk, Wv, bv, scale):
    N, D = x.shape
    Dh = D // 2
    B = 1000
    nb = N // B
    E = edge_index.shape[1]
    R, W = 2000, E // 2000
    BR = R // nb
    ein = edge_index.reshape(2, R, W)
    w_spec = pl.BlockSpec((Dh, D), lambda i, h: (h, 0))
    b_spec = pl.BlockSpec((1, 1, Dh), lambda i, h: (h, 0, 0))
    q2, kv2, p2 = pl.pallas_call(
        functools.partial(_qkv_body, scale),
        grid=(nb, 2),
        in_specs=[
            pl.BlockSpec((B, D), lambda i, h: (i, 0)),
            w_spec, w_spec, w_spec, b_spec, b_spec, b_spec,
            pl.BlockSpec((2, BR, W), lambda i, h: (0, i, 0)),
        ],
        out_specs=[
            pl.BlockSpec((B, Dh), lambda i, h: (h * nb + i, 0)),
            pl.BlockSpec((B, D), lambda i, h: (h * nb + i, 0)),
            pl.BlockSpec((BR, W), lambda i, h: (i, 0)),
        ],
        out_shape=[
            jax.ShapeDtypeStruct((2 * N, Dh), jnp.bfloat16),
            jax.ShapeDtypeStruct((2 * N, D), jnp.bfloat16),
            jax.ShapeDtypeStruct((R, W), jnp.int32),
        ],
    )(x, Wq, Wk, Wv, bq.reshape(2, 1, Dh), bk.reshape(2, 1, Dh),
      bv.reshape(2, 1, Dh), ein)
    return q2, kv2, p2.reshape(E)


def _edge_body(nodes_n, chunk_c, chunks_n,
               q2, kv2, pk3, aout,
               pk0, pk1, pk2_, ki0, ki1, ki2, qi0, qi1, qi2,
               ds0, ds1, kv0, kv1, kv2_, qb0, qb1, qb2, ob0, ob1,
               acc, is0, is1, is2, gs0, gs1, gs2, ss0, ss1):
    N = nodes_n
    C = chunk_c
    D = ob0.shape[1]
    NW = 10
    rows_per = N // NW

    c = lax.axis_index("c")
    s = lax.axis_index("s")
    cN = c * N

    pkc = (pk0, pk1, pk2_)
    ki = (ki0, ki1, ki2)
    qi = (qi0, qi1, qi2)
    dsc = (ds0, ds1)
    kvb = (kv0, kv1, kv2_)
    qb = (qb0, qb1, qb2)
    ob = (ob0, ob1)
    isem = (is0, is1, is2)
    gs = (gs0, gs1, gs2)
    ss = (ss0, ss1)

    def zfill(i, _):
        for j in range(D // 16):
            ob0[i, pl.ds(j * 16, 16)] = jnp.zeros((16,), jnp.float32)
        return 0
    lax.fori_loop(0, C, zfill, 0)

    @pl.when(s < NW)
    def _zero():
        base = s * rows_per
        for r in range(rows_per // C):
            pltpu.sync_copy(ob0, acc.at[pl.ds(base + r * C, C)])
        rem = rows_per % C
        if rem:
            pltpu.sync_copy(ob0.at[pl.ds(0, rem)],
                            acc.at[pl.ds(base + rows_per - rem, rem)])
    plsc.subcore_barrier()

    def issue_rows(ch, b):
        for j in range(C // 16):
            sl = pl.ds(j * 16, 16)
            pe = pkc[b][sl]
            ki[b][sl] = jnp.bitwise_and(pe, 0xFFFF) + cN
            qi[b][sl] = jnp.right_shift(pe, 16) + cN
        pltpu.async_copy(kv2.at[ki[b]], kvb[b], gs[b])
        pltpu.async_copy(q2.at[qi[b]], qb[b], gs[b])

    himask = jnp.int32(-65536)

    def expand(w):
        ev = plsc.bitcast(jnp.left_shift(w, 16), jnp.float32)
        od = plsc.bitcast(jnp.bitwise_and(w, himask), jnp.float32)
        return ev, od

    def slot_work(g, b, b2):
        pltpu.make_async_copy(kv2.at[ki[b]], kvb[b], gs[b]).wait()
        pltpu.make_async_copy(q2.at[qi[b]], qb[b], gs[b]).wait()

        @plsc.parallel_loop(0, C, unroll=4)
        def edge(e):
            for grp in range(D // 64):
                kw = plsc.bitcast(kvb[b][e, pl.ds(grp * 32, 32)], jnp.int32)
                qw = plsc.bitcast(qb[b][e, pl.ds(grp * 32, 32)], jnp.int32)
                vw = plsc.bitcast(kvb[b][e, pl.ds(D // 2 + grp * 32, 32)], jnp.int32)
                ke, ko = expand(kw)
                qe, qo = expand(qw)
                ve, vo = expand(vw)
                me = jnp.exp(ke * qe)
                mo = jnp.exp(ko * qo)
                ob[b2][e, pl.ds(grp * 32, 16)] = me
                ob[b2][e, pl.ds(grp * 32 + 16, 16)] = mo
                ob[b2][e, pl.ds(D // 2 + grp * 32, 16)] = me * ve
                ob[b2][e, pl.ds(D // 2 + grp * 32 + 16, 16)] = mo * vo

        for j in range(C // 16):
            sl = pl.ds(j * 16, 16)
            dsc[b2][sl] = jnp.right_shift(pkc[b][sl], 16)
        pltpu.async_copy(ob[b2], acc.at[dsc[b2]], ss[b2], add=True)

        pb = (b2 + 1) % 2
        @pl.when(g >= 1)
        def _drain_scatter():
            pltpu.make_async_copy(ob[pb], acc.at[dsc[pb]], ss[pb]).wait()

        nb = (b + 2) % 3
        @pl.when(g + 2 < chunks_n)
        def _rows_ahead():
            @pl.when(g >= 1)
            def _drain_idx():
                pltpu.make_async_copy(pk3.at[s, 0], pkc[nb], isem[nb]).wait()
            issue_rows(g + 2, nb)

        @pl.when(g + 3 < chunks_n)
        def _idx_ahead():
            pltpu.async_copy(pk3.at[s, g + 3], pkc[b], isem[b])

    pltpu.sync_copy(pk3.at[s, 0], pk0)
    pltpu.sync_copy(pk3.at[s, 1], pk1)
    pltpu.sync_copy(pk3.at[s, 2], pk2_)
    issue_rows(0, 0)
    issue_rows(1, 1)

    def pipe(i, _):
        for t in range(6):
            slot_work(6 * i + t, t % 3, t % 2)
        return 0
    lax.fori_loop(0, chunks_n // 6, pipe, 0)
    for g in range(chunks_n - chunks_n % 6, chunks_n):
        slot_work(g, g % 3, g % 2)

    pltpu.make_async_copy(ob[(chunks_n - 1) % 2],
                          acc.at[dsc[(chunks_n - 1) % 2]],
                          ss[(chunks_n - 1) % 2]).wait()
    plsc.subcore_barrier()

    @pl.when(s < NW)
    def _writeout():
        wbase = s * rows_per
        pltpu.sync_copy(acc.at[pl.ds(wbase, rows_per)],
                        aout.at[pl.ds(cN + wbase, rows_per)])


def _edge_pass(q2, kv2, packed, N, D):
    E = packed.shape[0]
    NS = 16
    Es = E // NS
    C = 80
    nch = Es // C
    Dh = D // 2
    mesh = plsc.VectorSubcoreMesh(core_axis_name="c", subcore_axis_name="s")
    idx = pltpu.VMEM((C,), jnp.int32)
    f = pl.kernel(
        functools.partial(_edge_body, N, C, nch),
        out_type=jax.ShapeDtypeStruct((2 * N, D), jnp.float32),
        mesh=mesh,
        scratch_types=[
            idx, idx, idx,
            idx, idx, idx,
            idx, idx, idx,
            idx, idx,
            pltpu.VMEM((C, D), jnp.bfloat16),
            pltpu.VMEM((C, D), jnp.bfloat16),
            pltpu.VMEM((C, D), jnp.bfloat16),
            pltpu.VMEM((C, Dh), jnp.bfloat16),
            pltpu.VMEM((C, Dh), jnp.bfloat16),
            pltpu.VMEM((C, Dh), jnp.bfloat16),
            pltpu.VMEM((C, D), jnp.float32),
            pltpu.VMEM((C, D), jnp.float32),
            pltpu.VMEM_SHARED((N, D), jnp.float32),
            pltpu.SemaphoreType.DMA,
            pltpu.SemaphoreType.DMA,
            pltpu.SemaphoreType.DMA,
            pltpu.SemaphoreType.DMA,
            pltpu.SemaphoreType.DMA,
            pltpu.SemaphoreType.DMA,
            pltpu.SemaphoreType.DMA,
            pltpu.SemaphoreType.DMA,
        ],
        compiler_params=pltpu.CompilerParams(use_tc_tiling_on_sc=False, needs_layout_passes=False),
    )
    return f(q2, kv2, packed.reshape(NS, nch, C))


def _out_body(a, wo, bo, out):
    Dh = a.shape[2] // 2
    al = a[0]
    ah = a[1]
    zl = al[:, :Dh]
    zh = ah[:, :Dh]
    rl = al[:, Dh:] / jnp.where(zl == 0.0, 1.0, zl)
    rh = ah[:, Dh:] / jnp.where(zh == 0.0, 1.0, zh)
    r = jnp.concatenate([rl, rh], axis=1)
    dn = (((1,), (1,)), ((), ()))
    out[...] = lax.dot_general(r, wo[...], dn, preferred_element_type=jnp.float32) + bo[0]


def _out_proj(A, Wo, bo, N, D):
    B = 2000
    nb = N // B
    a3 = A.reshape(2, N, D)
    return pl.pallas_call(
        _out_body,
        grid=(nb,),
        in_specs=[
            pl.BlockSpec((2, B, D), lambda i: (0, i, 0)),
            pl.BlockSpec((D, D), lambda i: (0, 0)),
            pl.BlockSpec((1, D), lambda i: (0, 0)),
        ],
        out_specs=pl.BlockSpec((B, D), lambda i: (i, 0)),
        out_shape=jax.ShapeDtypeStruct((N, D), jnp.float32),
    )(a3, Wo, bo.reshape(1, D))


def kernel(x, edge_index, Wq, bq, Wk, bk, Wv, bv, Wo, bo):
    N, D = x.shape
    dk = D // H
    scale = 1.0 / math.sqrt(dk)
    q2, kv2, packed = _qkv_proj(x, edge_index, Wq, bq, Wk, bk, Wv, bv, scale)
    A = _edge_pass(q2, kv2, packed, N, D)
    perm64 = [g * 32 + u for g in range(2) for u in
              list(range(0, 32, 2)) + list(range(1, 32, 2))]
    perm = jnp.array(perm64 + [64 + p for p in perm64], dtype=jnp.int32)
    return _out_proj(A, Wo[:, perm], bo, N, D)


# speedup vs baseline: 3.7030x; 1.0202x over previous
import functools
import math

import jax
import jax.numpy as jnp
from jax import lax
from jax.experimental import pallas as pl
from jax.experimental.pallas import tpu as pltpu
from jax.experimental.pallas import tpu_sc as plsc

H = 8


def _qkv_body(scale, xb, wq, wk, wv, bq, bk, bv, ei, qo, kvo, po):
    x = xb[...]
    dn = (((1,), (1,)), ((), ()))
    q = lax.dot_general(x, wq[...], dn, preferred_element_type=jnp.float32) + bq[0]
    qo[...] = q.astype(jnp.bfloat16)
    k = (lax.dot_general(x, wk[...], dn, preferred_element_type=jnp.float32) + bk[0]) * scale
    v = lax.dot_general(x, wv[...], dn, preferred_element_type=jnp.float32) + bv[0]
    kvo[...] = jnp.concatenate([k, v], axis=1).astype(jnp.bfloat16)
    e = ei[...]
    po[...] = jnp.bitwise_or(jnp.left_shift(e[1], 16), e[0])


def _qkv_proj(x, edge_index, Wq, bq, Wk, bk, Wv, bv, scale):
    N, D = x.shape
    Dh = D // 2
    B = 2000
    nb = N // B
    E = edge_index.shape[1]
    R, W = 2000, E // 2000
    BR = R // nb
    ein = edge_index.reshape(2, R, W)
    w_spec = pl.BlockSpec((Dh, D), lambda i, h: (h, 0))
    b_spec = pl.BlockSpec((1, 1, Dh), lambda i, h: (h, 0, 0))
    q2, kv2, p2 = pl.pallas_call(
        functools.partial(_qkv_body, scale),
        grid=(nb, 2),
        in_specs=[
            pl.BlockSpec((B, D), lambda i, h: (i, 0)),
            w_spec, w_spec, w_spec, b_spec, b_spec, b_spec,
            pl.BlockSpec((2, BR, W), lambda i, h: (0, i, 0)),
        ],
        out_specs=[
            pl.BlockSpec((B, Dh), lambda i, h: (h * nb + i, 0)),
            pl.BlockSpec((B, D), lambda i, h: (h * nb + i, 0)),
            pl.BlockSpec((BR, W), lambda i, h: (i, 0)),
        ],
        out_shape=[
            jax.ShapeDtypeStruct((2 * N, Dh), jnp.bfloat16),
            jax.ShapeDtypeStruct((2 * N, D), jnp.bfloat16),
            jax.ShapeDtypeStruct((R, W), jnp.int32),
        ],
    )(x, Wq, Wk, Wv, bq.reshape(2, 1, Dh), bk.reshape(2, 1, Dh),
      bv.reshape(2, 1, Dh), ein)
    return q2, kv2, p2.reshape(E)


def _edge_body(nodes_n, chunk_c, chunks_n,
               q2, kv2, pk3, aout,
               pk0, pk1, pk2_, ki0, ki1, ki2, qi0, qi1, qi2,
               ds0, ds1, kv0, kv1, kv2_, qb0, qb1, qb2, ob0, ob1,
               acc, is0, is1, is2, gs0, gs1, gs2, ss0, ss1):
    N = nodes_n
    C = chunk_c
    D = ob0.shape[1]
    NW = 10
    rows_per = N // NW

    c = lax.axis_index("c")
    s = lax.axis_index("s")
    cN = c * N

    pkc = (pk0, pk1, pk2_)
    ki = (ki0, ki1, ki2)
    qi = (qi0, qi1, qi2)
    dsc = (ds0, ds1)
    kvb = (kv0, kv1, kv2_)
    qb = (qb0, qb1, qb2)
    ob = (ob0, ob1)
    isem = (is0, is1, is2)
    gs = (gs0, gs1, gs2)
    ss = (ss0, ss1)

    def zfill(i, _):
        for j in range(D // 16):
            ob0[i, pl.ds(j * 16, 16)] = jnp.zeros((16,), jnp.float32)
        return 0
    lax.fori_loop(0, C, zfill, 0)

    @pl.when(s < NW)
    def _zero():
        base = s * rows_per
        for r in range(rows_per // C):
            pltpu.sync_copy(ob0, acc.at[pl.ds(base + r * C, C)])
        rem = rows_per % C
        if rem:
            pltpu.sync_copy(ob0.at[pl.ds(0, rem)],
                            acc.at[pl.ds(base + rows_per - rem, rem)])
    plsc.subcore_barrier()

    def issue_rows(ch, b):
        for j in range(C // 16):
            sl = pl.ds(j * 16, 16)
            pe = pkc[b][sl]
            ki[b][sl] = jnp.bitwise_and(pe, 0xFFFF) + cN
            qi[b][sl] = jnp.right_shift(pe, 16) + cN
        pltpu.async_copy(kv2.at[ki[b]], kvb[b], gs[b])
        pltpu.async_copy(q2.at[qi[b]], qb[b], gs[b])

    himask = jnp.int32(-65536)

    def expand(w):
        ev = plsc.bitcast(jnp.left_shift(w, 16), jnp.float32)
        od = plsc.bitcast(jnp.bitwise_and(w, himask), jnp.float32)
        return ev, od

    def slot_work(g, b, b2):
        pltpu.make_async_copy(kv2.at[ki[b]], kvb[b], gs[b]).wait()
        pltpu.make_async_copy(q2.at[qi[b]], qb[b], gs[b]).wait()

        @plsc.parallel_loop(0, C, unroll=4)
        def edge(e):
            for grp in range(D // 64):
                kw = plsc.bitcast(kvb[b][e, pl.ds(grp * 32, 32)], jnp.int32)
                qw = plsc.bitcast(qb[b][e, pl.ds(grp * 32, 32)], jnp.int32)
                vw = plsc.bitcast(kvb[b][e, pl.ds(D // 2 + grp * 32, 32)], jnp.int32)
                ke, ko = expand(kw)
                qe, qo = expand(qw)
                ve, vo = expand(vw)
                me = jnp.exp(ke * qe)
                mo = jnp.exp(ko * qo)
                ob[b2][e, pl.ds(grp * 32, 16)] = me
                ob[b2][e, pl.ds(grp * 32 + 16, 16)] = mo
                ob[b2][e, pl.ds(D // 2 + grp * 32, 16)] = me * ve
                ob[b2][e, pl.ds(D // 2 + grp * 32 + 16, 16)] = mo * vo

        for j in range(C // 16):
            sl = pl.ds(j * 16, 16)
            dsc[b2][sl] = jnp.right_shift(pkc[b][sl], 16)
        pltpu.async_copy(ob[b2], acc.at[dsc[b2]], ss[b2], add=True)

        pb = (b2 + 1) % 2
        @pl.when(g >= 1)
        def _drain_scatter():
            pltpu.make_async_copy(ob[pb], acc.at[dsc[pb]], ss[pb]).wait()

        nb = (b + 2) % 3
        @pl.when(g + 2 < chunks_n)
        def _rows_ahead():
            @pl.when(g >= 1)
            def _drain_idx():
                pltpu.make_async_copy(pk3.at[s, 0], pkc[nb], isem[nb]).wait()
            issue_rows(g + 2, nb)

        @pl.when(g + 3 < chunks_n)
        def _idx_ahead():
            pltpu.async_copy(pk3.at[s, g + 3], pkc[b], isem[b])

    pltpu.sync_copy(pk3.at[s, 0], pk0)
    pltpu.sync_copy(pk3.at[s, 1], pk1)
    pltpu.sync_copy(pk3.at[s, 2], pk2_)
    issue_rows(0, 0)
    issue_rows(1, 1)

    def pipe(i, _):
        for t in range(6):
            slot_work(6 * i + t, t % 3, t % 2)
        return 0
    lax.fori_loop(0, chunks_n // 6, pipe, 0)
    for g in range(chunks_n - chunks_n % 6, chunks_n):
        slot_work(g, g % 3, g % 2)

    pltpu.make_async_copy(ob[(chunks_n - 1) % 2],
                          acc.at[dsc[(chunks_n - 1) % 2]],
                          ss[(chunks_n - 1) % 2]).wait()
    plsc.subcore_barrier()

    @pl.when(s < NW)
    def _writeout():
        wbase = s * rows_per
        pltpu.sync_copy(acc.at[pl.ds(wbase, rows_per)],
                        aout.at[pl.ds(cN + wbase, rows_per)])


def _edge_pass(q2, kv2, packed, N, D):
    E = packed.shape[0]
    NS = 16
    Es = E // NS
    C = 80
    nch = Es // C
    Dh = D // 2
    mesh = plsc.VectorSubcoreMesh(core_axis_name="c", subcore_axis_name="s")
    idx = pltpu.VMEM((C,), jnp.int32)
    f = pl.kernel(
        functools.partial(_edge_body, N, C, nch),
        out_type=jax.ShapeDtypeStruct((2 * N, D), jnp.float32),
        mesh=mesh,
        scratch_types=[
            idx, idx, idx,
            idx, idx, idx,
            idx, idx, idx,
            idx, idx,
            pltpu.VMEM((C, D), jnp.bfloat16),
            pltpu.VMEM((C, D), jnp.bfloat16),
            pltpu.VMEM((C, D), jnp.bfloat16),
            pltpu.VMEM((C, Dh), jnp.bfloat16),
            pltpu.VMEM((C, Dh), jnp.bfloat16),
            pltpu.VMEM((C, Dh), jnp.bfloat16),
            pltpu.VMEM((C, D), jnp.float32),
            pltpu.VMEM((C, D), jnp.float32),
            pltpu.VMEM_SHARED((N, D), jnp.float32),
            pltpu.SemaphoreType.DMA,
            pltpu.SemaphoreType.DMA,
            pltpu.SemaphoreType.DMA,
            pltpu.SemaphoreType.DMA,
            pltpu.SemaphoreType.DMA,
            pltpu.SemaphoreType.DMA,
            pltpu.SemaphoreType.DMA,
            pltpu.SemaphoreType.DMA,
        ],
        compiler_params=pltpu.CompilerParams(use_tc_tiling_on_sc=False, needs_layout_passes=False),
    )
    return f(q2, kv2, packed.reshape(NS, nch, C))


def _out_body(a, wo, bo, out):
    Dh = a.shape[2] // 2
    al = a[0]
    ah = a[1]
    zl = al[:, :Dh]
    zh = ah[:, :Dh]
    rl = al[:, Dh:] / jnp.where(zl == 0.0, 1.0, zl)
    rh = ah[:, Dh:] / jnp.where(zh == 0.0, 1.0, zh)
    r = jnp.concatenate([rl, rh], axis=1)
    dn = (((1,), (1,)), ((), ()))
    out[...] = lax.dot_general(r, wo[...], dn, preferred_element_type=jnp.float32) + bo[0]


def _out_proj(A, Wo, bo, N, D):
    B = 2000
    nb = N // B
    a3 = A.reshape(2, N, D)
    return pl.pallas_call(
        _out_body,
        grid=(nb,),
        in_specs=[
            pl.BlockSpec((2, B, D), lambda i: (0, i, 0)),
            pl.BlockSpec((D, D), lambda i: (0, 0)),
            pl.BlockSpec((1, D), lambda i: (0, 0)),
        ],
        out_specs=pl.BlockSpec((B, D), lambda i: (i, 0)),
        out_shape=jax.ShapeDtypeStruct((N, D), jnp.float32),
    )(a3, Wo, bo.reshape(1, D))


def kernel(x, edge_index, Wq, bq, Wk, bk, Wv, bv, Wo, bo):
    N, D = x.shape
    dk = D // H
    scale = 1.0 / math.sqrt(dk)
    q2, kv2, packed = _qkv_proj(x, edge_index, Wq, bq, Wk, bk, Wv, bv, scale)
    A = _edge_pass(q2, kv2, packed, N, D)
    perm64 = [g * 32 + u for g in range(2) for u in
              list(range(0, 32, 2)) + list(range(1, 32, 2))]
    perm = jnp.array(perm64 + [64 + p for p in perm64], dtype=jnp.int32)
    return _out_proj(A, Wo[:, perm], bo, N, D)
